# baseline (speedup 1.0000x reference)
import functools

import jax
import jax.numpy as jnp
from jax import lax
from jax.experimental import pallas as pl
from jax.experimental.pallas import tpu as pltpu
from jax.experimental.pallas import tpu_sc as plsc

VOCAB_ROWS = 32
EMBED_DIM = 64
BATCH = 4096
SEQ = 200
TOTAL = BATCH * SEQ

_info = plsc.get_sparse_core_info()
_NC = _info.num_cores
_NS = _info.num_subcores
_L = _info.num_lanes
CHUNK = 128
N_CHUNKS = TOTAL // CHUNK
NGROUP = CHUNK // _L
SHALF = 11
N_STILE = SHALF * _NC
N_VTILE = (_NS - SHALF) * _NC
SPT = 240
CPT = (N_CHUNKS - N_STILE * SPT) // N_VTILE
C_BASE = N_STILE * SPT
NBUF = 4
SKEW = 2


def _make_kernel():
    mesh = plsc.VectorSubcoreMesh(core_axis_name="c", subcore_axis_name="s")

    @functools.partial(
        pl.kernel,
        mesh=mesh,
        out_type=jax.ShapeDtypeStruct((TOTAL, EMBED_DIM), jnp.float32),
        compiler_params=pltpu.CompilerParams(
            use_tc_tiling_on_sc=False, needs_layout_passes=False),
        scratch_types=[
            pltpu.VMEM((SPT * CHUNK,), jnp.int32),
            pltpu.VMEM((NBUF, CHUNK, EMBED_DIM), jnp.float32),
            pltpu.VMEM((2, CHUNK, EMBED_DIM), jnp.float32),
            pltpu.VMEM((VOCAB_ROWS, EMBED_DIM), jnp.float32),
            pltpu.VMEM_SHARED((VOCAB_ROWS, EMBED_DIM), jnp.float32),
        ]
        + [pltpu.SemaphoreType.DMA] * 10,
    )
    def k(idx_hbm, table_hbm, out_hbm, idx_v, s_rows, c_rows, table_v,
          table_sh, g0, g1, g2, g3, o0, o1, o2, o3, oc0, oc1):
        gsem = [g0, g1, g2, g3]
        osem = [o0, o1, o2, o3]
        ocsem = [oc0, oc1]
        sid = lax.axis_index("s")
        cid = lax.axis_index("c")

        @pl.when(sid == 0)
        def _():
            pltpu.sync_copy(table_hbm, table_sh)

        plsc.subcore_barrier()

        lanes = lax.iota(jnp.int32, _L)

        @pl.when(sid < SHALF)
        def _stream_role():
            stid = sid * _NC + cid
            first = stid * SPT
            pltpu.sync_copy(idx_hbm.at[pl.ds(first * CHUNK, SPT * CHUNK)],
                            idx_v)

            def sg(q, b, start):
                cp = pltpu.make_async_copy(
                    table_sh.at[idx_v.at[pl.ds(q * CHUNK, CHUNK)]],
                    s_rows.at[b], gsem[b])
                cp.start() if start else cp.wait()

            def sw(q, b, start):
                cp = pltpu.make_async_copy(
                    s_rows.at[b],
                    out_hbm.at[pl.ds((first + q) * CHUNK, CHUNK)], osem[b])
                cp.start() if start else cp.wait()

            for b in range(NBUF):
                sg(b, b, True)
            for b in range(SKEW):
                sg(b, b, False)
                sw(b, b, True)

            def body(i, carry):
                qb = i * NBUF
                for b in range(NBUF):
                    q = qb + b
                    sw(q - NBUF, b, False)
                    sg(q, b, True)
                    qw = q - SKEW
                    bw = (b + NBUF - SKEW) % NBUF
                    sg(qw, bw, False)
                    sw(qw, bw, True)
                return carry

            lax.fori_loop(1, SPT // NBUF, body, 0)

            lastq = SPT - NBUF
            for b in range(SKEW, NBUF):
                sg(lastq + b, b, False)
                sw(lastq + b, b, True)
            for b in range(NBUF):
                sw(lastq + b, b, False)

        @pl.when(sid >= SHALF)
        def _vector_role():
            ctid = (sid - SHALF) * _NC + cid
            first = C_BASE + ctid * CPT
            pltpu.sync_copy(table_hbm, table_v)
            pltpu.sync_copy(idx_hbm.at[pl.ds(first * CHUNK, CPT * CHUNK)],
                            idx_v.at[pl.ds(0, CPT * CHUNK)])

            def comp(c, u):
                buf = c_rows.at[u]

                @plsc.parallel_loop(0, NGROUP, unroll=1)
                def group(g):
                    idx_vec = idx_v[pl.ds(c * CHUNK + g * _L, _L)]
                    rowv = g * _L + lanes
                    for cc in range(EMBED_DIM):
                        cvec = (lanes + cc) & (EMBED_DIM - 1)
                        v = plsc.load_gather(table_v, [idx_vec, cvec])
                        plsc.store_scatter(buf, [rowv, cvec], v)

            def cw(c, u, start):
                cp = pltpu.make_async_copy(
                    c_rows.at[u],
                    out_hbm.at[pl.ds((first + c) * CHUNK, CHUNK)], ocsem[u])
                cp.start() if start else cp.wait()

            for u in range(2):
                comp(u, u)
                cw(u, u, True)

            def body(i, carry):
                cb = i * 2
                for u in range(2):
                    c = cb + u
                    cw(c - 2, u, False)
                    comp(c, u)
                    cw(c, u, True)
                return carry

            lax.fori_loop(1, CPT // 2, body, 0)

            for u in range(2):
                cw(CPT - 2 + u, u, False)

    return k


_sc_gather = _make_kernel()


def kernel(inputs, embedding_table):
    idx = inputs.reshape(TOTAL)
    out = _sc_gather(idx, embedding_table)
    return out.reshape(BATCH, SEQ, EMBED_DIM)


# speedup vs baseline: 1.0426x; 1.0426x over previous
import functools

import jax
import jax.numpy as jnp
from jax import lax
from jax.experimental import pallas as pl
from jax.experimental.pallas import tpu as pltpu
from jax.experimental.pallas import tpu_sc as plsc

VOCAB_ROWS = 32
EMBED_DIM = 64
BATCH = 4096
SEQ = 200
TOTAL = BATCH * SEQ

_info = plsc.get_sparse_core_info()
_NC = _info.num_cores
_NS = _info.num_subcores
_NW = _NC * _NS
PER_W = TOTAL // _NW
CHUNK = 128
N_CHUNKS = PER_W // CHUNK
NBUF = 4
SKEW = 2


def _make_kernel():
    mesh = plsc.VectorSubcoreMesh(core_axis_name="c", subcore_axis_name="s")

    @functools.partial(
        pl.kernel,
        mesh=mesh,
        out_type=jax.ShapeDtypeStruct((TOTAL, EMBED_DIM), jnp.float32),
        compiler_params=pltpu.CompilerParams(use_tc_tiling_on_sc=False),
        scratch_types=[
            pltpu.VMEM((PER_W,), jnp.int32),
            pltpu.VMEM((NBUF, CHUNK, EMBED_DIM), jnp.float32),
            pltpu.VMEM_SHARED((VOCAB_ROWS, EMBED_DIM), jnp.float32),
        ]
        + [pltpu.SemaphoreType.DMA] * (2 * NBUF),
    )
    def k(idx_hbm, table_hbm, out_hbm, idx_v, rows, table_sh,
          g0, g1, g2, g3, o0, o1, o2, o3):
        gsem = [g0, g1, g2, g3]
        osem = [o0, o1, o2, o3]
        wid = lax.axis_index("s") * _NC + lax.axis_index("c")
        base = wid * PER_W

        @pl.when(lax.axis_index("s") == 0)
        def _():
            pltpu.sync_copy(table_hbm, table_sh)

        pltpu.sync_copy(idx_hbm.at[pl.ds(base, PER_W)], idx_v)
        plsc.subcore_barrier()

        def sg(q, b, start):
            cp = pltpu.make_async_copy(
                table_sh.at[idx_v.at[pl.ds(q * CHUNK, CHUNK)]],
                rows.at[b], gsem[b])
            cp.start() if start else cp.wait()

        def sw(q, b, start):
            cp = pltpu.make_async_copy(
                rows.at[b],
                out_hbm.at[pl.ds(base + q * CHUNK, CHUNK)], osem[b])
            cp.start() if start else cp.wait()

        for b in range(NBUF):
            sg(b, b, True)
        for b in range(SKEW):
            sg(b, b, False)
            sw(b, b, True)

        def body(i, carry):
            qb = i * NBUF
            for b in range(NBUF):
                q = qb + b
                sw(q - NBUF, b, False)
                sg(q, b, True)
                qw = q - SKEW
                bw = (b + NBUF - SKEW) % NBUF
                sg(qw, bw, False)
                sw(qw, bw, True)
            return carry

        lax.fori_loop(1, N_CHUNKS // NBUF, body, 0)

        lastq = N_CHUNKS - NBUF
        for b in range(SKEW, NBUF):
            sg(lastq + b, b, False)
            sw(lastq + b, b, True)
        for b in range(NBUF):
            sw(lastq + b, b, False)

    return k


_sc_gather = _make_kernel()


def kernel(inputs, embedding_table):
    idx = inputs.reshape(TOTAL)
    out = _sc_gather(idx, embedding_table)
    return out.reshape(BATCH, SEQ, EMBED_DIM)
